# baseline (speedup 1.0000x reference)
import functools

import jax
import jax.numpy as jnp
from jax import lax
from jax.experimental import pallas as pl
from jax.experimental.pallas import tpu as pltpu
from jax.experimental.pallas import tpu_sc as plsc

DIM = 64

_info = plsc.get_sparse_core_info()
_NC, _NS = _info.num_cores, _info.num_subcores
_NW = _NC * _NS

_CHUNK = 128
_NBUF = 4


@jax.jit
def _gather_pallas(fused, idx):
    n = idx.shape[0]
    b_per_w = n // _NW
    n_stages = b_per_w // _CHUNK

    mesh = plsc.VectorSubcoreMesh(core_axis_name="c", subcore_axis_name="s")

    @functools.partial(
        pl.kernel,
        mesh=mesh,
        out_type=jax.ShapeDtypeStruct((n, 2 * DIM), jnp.float32),
        scratch_types=[
            pltpu.VMEM((b_per_w,), jnp.int32),
            pltpu.VMEM((_NBUF * _CHUNK, 2 * DIM), jnp.float32),
            pltpu.SemaphoreType.DMA,
            pltpu.SemaphoreType.DMA,
        ],
    )
    def k(fused_hbm, idx_hbm, out_hbm, idx_v, rows_v, gsem, osem):
        wid = lax.axis_index("s") * _NC + lax.axis_index("c")
        base = wid * b_per_w
        pltpu.sync_copy(idx_hbm.at[pl.ds(base, b_per_w)], idx_v)

        def buf(s):
            return rows_v.at[pl.ds((s % _NBUF) * _CHUNK, _CHUNK)]

        def fire(s):
            idx_c = idx_v.at[pl.ds(s * _CHUNK, _CHUNK)]
            pltpu.async_copy(fused_hbm.at[idx_c], buf(s), gsem)

        def drain(sem, s):
            pltpu.make_async_copy(fused_hbm.at[pl.ds(0, _CHUNK)], buf(s), sem).wait()

        for s in range(min(_NBUF, n_stages)):
            fire(s)
        for s in range(n_stages):
            drain(gsem, s)
            pltpu.async_copy(buf(s), out_hbm.at[pl.ds(base + s * _CHUNK, _CHUNK)], osem)
            if s >= 1 and s - 1 + _NBUF < n_stages:
                drain(osem, s - 1)
                fire(s - 1 + _NBUF)
        for s in range(max(0, n_stages - _NBUF), n_stages):
            drain(osem, s)

    return k(fused, idx)


def kernel(x, position_ids, cos_cached, sin_cached):
    b, s = position_ids.shape
    idx = position_ids.reshape(-1).astype(jnp.int32)
    fused = jnp.concatenate([cos_cached, sin_cached], axis=1)
    out = _gather_pallas(fused, idx)
    cos = out[:, :DIM].reshape(b, s, DIM).astype(x.dtype)
    sin = out[:, DIM:].reshape(b, s, DIM).astype(x.dtype)
    return (cos, sin)
